# baseline (speedup 1.0000x reference)
import functools

import jax
import jax.numpy as jnp
from jax import lax
from jax.experimental import pallas as pl
from jax.experimental.pallas import tpu as pltpu
from jax.experimental.pallas import tpu_sc as plsc

B = 16384
D = 16
N_ROWS = 1000000
NUM_CORES = 2
NUM_SUBCORES = 16
NW = NUM_CORES * NUM_SUBCORES
BPW = B // NW
LANES = 16
GROUPS = BPW // LANES

DEPAD_ROWS = 8000
DEPAD_STEPS = N_ROWS // DEPAD_ROWS

_mesh = plsc.VectorSubcoreMesh(core_axis_name="c", subcore_axis_name="s")


def _tc_depad_body(t_ref, out_ref):
    inb = t_ref[...].reshape(DEPAD_ROWS // 8, 8, D)
    parts = [inb[:, l, :] for l in range(8)]
    out_ref[...] = jnp.concatenate(parts, axis=1)


_tc_depad = pl.pallas_call(
    _tc_depad_body,
    grid=(DEPAD_STEPS,),
    in_specs=[pl.BlockSpec((DEPAD_ROWS, D), lambda i: (i, 0))],
    out_specs=pl.BlockSpec((DEPAD_ROWS // 8, 8 * D), lambda i: (i, 0)),
    out_shape=jax.ShapeDtypeStruct((N_ROWS // 8, 8 * D), jnp.float32),
)


CHUNK = 128
NCHUNKS = BPW // CHUNK
CGROUPS = CHUNK // LANES
BLK = 8 * D


@functools.partial(
    pl.kernel,
    mesh=_mesh,
    out_type=jax.ShapeDtypeStruct((B,), jnp.float32),
    scratch_types=[
        pltpu.VMEM((BPW,), jnp.int32),
        pltpu.VMEM((BPW,), jnp.int32),
        pltpu.VMEM((BPW,), jnp.int32),
        pltpu.VMEM((BPW,), jnp.int32),
        pltpu.VMEM((BPW,), jnp.int32),
        pltpu.VMEM((BPW,), jnp.int32),
        pltpu.VMEM((CHUNK, BLK), jnp.float32),
        pltpu.VMEM((CHUNK, BLK), jnp.float32),
        pltpu.VMEM((CHUNK, BLK), jnp.float32),
        pltpu.VMEM((BPW,), jnp.float32),
        pltpu.SemaphoreType.DMA,
    ],
    compiler_params=pltpu.CompilerParams(
        needs_layout_passes=False, use_tc_tiling_on_sc=False
    ),
)
def _sc_ppush(user_blk, item_blk, users, pos, neg, out,
              ui_v, pi_v, ni_v, ub_v, pb_v, nb_v,
              ur_v, pr_v, nr_v, acc_v, sem):
    wid = lax.axis_index("s") * NUM_CORES + lax.axis_index("c")
    base = pl.multiple_of(wid * BPW, BPW)

    pltpu.sync_copy(users.at[pl.ds(base, BPW)], ui_v)
    pltpu.sync_copy(pos.at[pl.ds(base, BPW)], pi_v)
    pltpu.sync_copy(neg.at[pl.ds(base, BPW)], ni_v)

    def build_body(g, carry):
        goff = pl.multiple_of(g * LANES, LANES)
        ub_v[pl.ds(goff, LANES)] = lax.shift_right_logical(
            ui_v[pl.ds(goff, LANES)], 3)
        pb_v[pl.ds(goff, LANES)] = lax.shift_right_logical(
            pi_v[pl.ds(goff, LANES)], 3)
        nb_v[pl.ds(goff, LANES)] = lax.shift_right_logical(
            ni_v[pl.ds(goff, LANES)], 3)
        return carry

    lax.fori_loop(0, GROUPS, build_body, 0)

    lane_iota = lax.iota(jnp.int32, LANES)

    def chunk_body(c, carry):
        coff = pl.multiple_of(c * CHUNK, CHUNK)
        cu = pltpu.async_copy(user_blk.at[ub_v.at[pl.ds(coff, CHUNK)]],
                              ur_v, sem)
        cp = pltpu.async_copy(item_blk.at[pb_v.at[pl.ds(coff, CHUNK)]],
                              pr_v, sem)
        cn = pltpu.async_copy(item_blk.at[nb_v.at[pl.ds(coff, CHUNK)]],
                              nr_v, sem)
        cu.wait()
        cp.wait()
        cn.wait()

        def group_body(g, carry2):
            goff = pl.multiple_of(g * LANES, LANES)
            pos_in_chunk = goff + lane_iota
            ucol = (ui_v[pl.ds(coff + goff, LANES)] & 7) << 4
            pcol = (pi_v[pl.ds(coff + goff, LANES)] & 7) << 4
            ncol = (ni_v[pl.ds(coff + goff, LANES)] & 7) << 4
            acc = jnp.zeros((LANES,), jnp.float32)
            for d in range(D):
                u = plsc.load_gather(ur_v, [pos_in_chunk, ucol + d])
                p = plsc.load_gather(pr_v, [pos_in_chunk, pcol + d])
                n = plsc.load_gather(nr_v, [pos_in_chunk, ncol + d])
                acc = acc + u * (p - n)
            acc_v[pl.ds(coff + goff, LANES)] = acc
            return carry2

        lax.fori_loop(0, CGROUPS, group_body, 0)
        return carry

    lax.fori_loop(0, NCHUNKS, chunk_body, 0)

    pltpu.sync_copy(acc_v, out.at[pl.ds(base, BPW)])


def kernel(users, pos_items, neg_items, user_emb, item_emb):
    user_blk = _tc_depad(user_emb)
    item_blk = _tc_depad(item_emb)
    return _sc_ppush(
        user_blk,
        item_blk,
        users.astype(jnp.int32),
        pos_items.astype(jnp.int32),
        neg_items.astype(jnp.int32),
    )


# speedup vs baseline: 1.7971x; 1.7971x over previous
import functools

import jax
import jax.numpy as jnp
from jax import lax
from jax.experimental import pallas as pl
from jax.experimental.pallas import tpu as pltpu
from jax.experimental.pallas import tpu_sc as plsc

B = 16384
D = 16
RPB = 8
NUM_CORES = 2
NUM_SUBCORES = 16
NW = NUM_CORES * NUM_SUBCORES
BPW = B // NW
LANES = 16
CHUNK = 32
NCHUNKS = BPW // CHUNK
CGROUPS = CHUNK // LANES

_mesh = plsc.VectorSubcoreMesh(core_axis_name="c", subcore_axis_name="s")


@functools.partial(
    pl.kernel,
    mesh=_mesh,
    out_type=jax.ShapeDtypeStruct((B,), jnp.float32),
    scratch_types=[
        pltpu.VMEM((BPW,), jnp.int32),
        pltpu.VMEM((BPW,), jnp.int32),
        pltpu.VMEM((BPW,), jnp.int32),
        pltpu.VMEM((CHUNK * RPB, D), jnp.float32),
        pltpu.VMEM((CHUNK * RPB, D), jnp.float32),
        pltpu.VMEM((CHUNK * RPB, D), jnp.float32),
        pltpu.VMEM((BPW,), jnp.float32),
        pltpu.SemaphoreType.DMA,
    ],
    compiler_params=pltpu.CompilerParams(
        needs_layout_passes=False, use_tc_tiling_on_sc=True
    ),
)
def _sc_ppush(user_emb, item_emb, users, pos, neg, out,
              ui_v, pi_v, ni_v,
              ur_v, pr_v, nr_v, acc_v, sem):
    wid = lax.axis_index("s") * NUM_CORES + lax.axis_index("c")
    base = pl.multiple_of(wid * BPW, BPW)

    pltpu.sync_copy(users.at[pl.ds(base, BPW)], ui_v)
    pltpu.sync_copy(pos.at[pl.ds(base, BPW)], pi_v)
    pltpu.sync_copy(neg.at[pl.ds(base, BPW)], ni_v)

    lane_iota = lax.iota(jnp.int32, LANES)

    def chunk_body(c, carry):
        coff = pl.multiple_of(c * CHUNK, CHUNK)

        def issue_body(j, carry2):
            joff = pl.multiple_of(j * LANES, LANES)
            ub16 = (ui_v[pl.ds(coff + joff, LANES)] >> 3) << 3
            pb16 = (pi_v[pl.ds(coff + joff, LANES)] >> 3) << 3
            nb16 = (ni_v[pl.ds(coff + joff, LANES)] >> 3) << 3
            for l in range(LANES):
                slot = pl.multiple_of((joff + l) * RPB, RPB)
                pltpu.async_copy(
                    user_emb.at[pl.ds(pl.multiple_of(ub16[l], RPB), RPB)],
                    ur_v.at[pl.ds(slot, RPB)], sem)
                pltpu.async_copy(
                    item_emb.at[pl.ds(pl.multiple_of(pb16[l], RPB), RPB)],
                    pr_v.at[pl.ds(slot, RPB)], sem)
                pltpu.async_copy(
                    item_emb.at[pl.ds(pl.multiple_of(nb16[l], RPB), RPB)],
                    nr_v.at[pl.ds(slot, RPB)], sem)
            return carry2

        lax.fori_loop(0, CHUNK // LANES, issue_body, 0)

        dummy = user_emb.at[pl.ds(0, CHUNK * RPB)]
        pltpu.make_async_copy(dummy, ur_v, sem).wait()
        pltpu.make_async_copy(dummy, pr_v, sem).wait()
        pltpu.make_async_copy(dummy, nr_v, sem).wait()

        def group_body(g, carry2):
            goff = pl.multiple_of(g * LANES, LANES)
            pos_in_chunk = goff + lane_iota
            urow = pos_in_chunk * RPB + (ui_v[pl.ds(coff + goff, LANES)] & 7)
            prow = pos_in_chunk * RPB + (pi_v[pl.ds(coff + goff, LANES)] & 7)
            nrow = pos_in_chunk * RPB + (ni_v[pl.ds(coff + goff, LANES)] & 7)
            acc = jnp.zeros((LANES,), jnp.float32)
            for d in range(D):
                dv = jnp.full((LANES,), d, jnp.int32)
                u = plsc.load_gather(ur_v, [urow, dv])
                p = plsc.load_gather(pr_v, [prow, dv])
                n = plsc.load_gather(nr_v, [nrow, dv])
                acc = acc + u * (p - n)
            acc_v[pl.ds(coff + goff, LANES)] = acc
            return carry2

        lax.fori_loop(0, CGROUPS, group_body, 0)
        return carry

    lax.fori_loop(0, NCHUNKS, chunk_body, 0)

    pltpu.sync_copy(acc_v, out.at[pl.ds(base, BPW)])


def kernel(users, pos_items, neg_items, user_emb, item_emb):
    return _sc_ppush(
        user_emb,
        item_emb,
        users.astype(jnp.int32),
        pos_items.astype(jnp.int32),
        neg_items.astype(jnp.int32),
    )
